# initial kernel scaffold (unmeasured)
import jax
import jax.numpy as jnp
from jax import lax
from jax.experimental import pallas as pl
from jax.experimental.pallas import tpu as pltpu

N_DEV = 8
SQ = 2048
D_MODEL = 1024
H_LOC = 8
DH = 128
H_COLS = H_LOC * DH
CHUNK = SQ // N_DEV
QBLK = 512
SCALE = 0.08838834764831843


def kernel(x, Wq, K_ext, V_ext, Wo):
    pos = lax.axis_index("i")
    Wq_loc = lax.dynamic_slice(Wq, (0, pos * H_COLS), (D_MODEL, H_COLS))
    Wo_loc = lax.dynamic_slice(Wo, (pos * H_COLS, 0), (H_COLS, D_MODEL))
    x2 = x[0]
    K = K_ext[0]
    V = V_ext[0]

    def body(x_ref, wq_ref, k_ref, v_ref, wo_ref, out_ref,
             q_ref, comm_ref, rs_send, rs_recv, ag_send, ag_recv):
        my = lax.axis_index("i")
        left = lax.rem(my + N_DEV - 1, N_DEV)
        right = lax.rem(my + 1, N_DEV)

        barrier = pltpu.get_barrier_semaphore()
        for nbr in (left, right):
            pl.semaphore_signal(barrier, inc=1, device_id=(nbr,),
                                device_id_type=pl.DeviceIdType.MESH)
        pl.semaphore_wait(barrier, 2)

        q_ref[...] = jnp.dot(x_ref[...], wq_ref[...],
                             preferred_element_type=jnp.float32)

        for h in range(H_LOC):
            k_h = k_ref[:, h, :]
            v_h = v_ref[:, h, :]
            wo_h = wo_ref[h * DH:(h + 1) * DH, :]
            for qb in range(SQ // QBLK):
                q0 = qb * QBLK
                q_blk = q_ref[q0:q0 + QBLK, h * DH:(h + 1) * DH]
                s = lax.dot_general(
                    q_blk, k_h, (((1,), (1,)), ((), ())),
                    preferred_element_type=jnp.float32) * SCALE
                rows = q0 + lax.broadcasted_iota(jnp.int32, (QBLK, SQ), 0)
                cols = lax.broadcasted_iota(jnp.int32, (QBLK, SQ), 1)
                qblock = rows // 64
                kblock = cols // 64
                mask = ((qblock == kblock) | (kblock == 0)
                        | (((qblock + kblock) % 3) == 0))
                s = jnp.where(mask, s, -1e9)
                m = jnp.max(s, axis=1, keepdims=True)
                w = jnp.exp(s - m)
                w = w / jnp.sum(w, axis=1, keepdims=True)
                ctx = jnp.dot(w, v_h, preferred_element_type=jnp.float32)
                contrib = jnp.dot(ctx, wo_h,
                                  preferred_element_type=jnp.float32)
                if h == 0:
                    out_ref[q0:q0 + QBLK, :] = contrib
                else:
                    out_ref[q0:q0 + QBLK, :] += contrib

        for s in range(N_DEV - 1):
            if s == 0:
                send_chunk = lax.rem(my + N_DEV, N_DEV)
                src = out_ref.at[pl.ds(send_chunk * CHUNK, CHUNK)]
            else:
                src = comm_ref.at[s - 1]
            rdma = pltpu.make_async_remote_copy(
                src_ref=src,
                dst_ref=comm_ref.at[s],
                send_sem=rs_send.at[s],
                recv_sem=rs_recv.at[s],
                device_id=(right,),
                device_id_type=pl.DeviceIdType.MESH,
            )
            rdma.start()
            rdma.wait()
            recv_chunk = lax.rem(my - s - 1 + 2 * N_DEV, N_DEV)
            comm_ref[s, :, :] = (comm_ref[s, :, :]
                                 + out_ref[pl.ds(recv_chunk * CHUNK, CHUNK), :])

        fin = lax.rem(my + 1, N_DEV)
        out_ref[pl.ds(fin * CHUNK, CHUNK), :] = comm_ref[N_DEV - 2, :, :]

        for s in range(N_DEV - 1):
            c = lax.rem(my + 1 - s + 2 * N_DEV, N_DEV)
            off = c * CHUNK
            rdma = pltpu.make_async_remote_copy(
                src_ref=out_ref.at[pl.ds(off, CHUNK)],
                dst_ref=out_ref.at[pl.ds(off, CHUNK)],
                send_sem=ag_send.at[s],
                recv_sem=ag_recv.at[s],
                device_id=(right,),
                device_id_type=pl.DeviceIdType.MESH,
            )
            rdma.start()
            rdma.wait()

    out = pl.pallas_call(
        body,
        out_shape=jax.ShapeDtypeStruct((SQ, D_MODEL), jnp.float32),
        in_specs=[pl.BlockSpec(memory_space=pltpu.VMEM)] * 5,
        out_specs=pl.BlockSpec(memory_space=pltpu.VMEM),
        scratch_shapes=[
            pltpu.VMEM((SQ, H_COLS), jnp.float32),
            pltpu.VMEM((N_DEV - 1, CHUNK, D_MODEL), jnp.float32),
            pltpu.SemaphoreType.DMA((N_DEV - 1,)),
            pltpu.SemaphoreType.DMA((N_DEV - 1,)),
            pltpu.SemaphoreType.DMA((N_DEV - 1,)),
            pltpu.SemaphoreType.DMA((N_DEV - 1,)),
        ],
        compiler_params=pltpu.CompilerParams(collective_id=0),
    )(x2, Wq_loc, K, V, Wo_loc)
    return out[None]


# baseline (device time: 356905 ns/iter reference)
import jax
import jax.numpy as jnp
from jax import lax
from jax.experimental import pallas as pl
from jax.experimental.pallas import tpu as pltpu

N_DEV = 8
SQ = 2048
D_MODEL = 1024
H_LOC = 8
DH = 128
H_COLS = H_LOC * DH
CHUNK = SQ // N_DEV
QBLK = 256
SCALE = 0.08838834764831843


def kernel(x, Wq, K_ext, V_ext, Wo):
    pos = lax.axis_index("i")
    Wq_loc = lax.dynamic_slice(Wq, (0, pos * H_COLS), (D_MODEL, H_COLS))
    Wo_loc = lax.dynamic_slice(Wo, (pos * H_COLS, 0), (H_COLS, D_MODEL))
    Q = x[0] @ Wq_loc
    K = K_ext[0]
    V = V_ext[0]

    def body(q_ref, k_ref, v_ref, wo_ref, out_ref,
             comm_ref, rs_send, rs_recv, ag_send, ag_recv):
        my = lax.axis_index("i")
        left = lax.rem(my + N_DEV - 1, N_DEV)
        right = lax.rem(my + 1, N_DEV)

        barrier = pltpu.get_barrier_semaphore()
        for nbr in (left, right):
            pl.semaphore_signal(barrier, inc=1, device_id=(nbr,),
                                device_id_type=pl.DeviceIdType.MESH)
        pl.semaphore_wait(barrier, 2)

        for h in range(H_LOC):
            k_h = k_ref[:, h, :]
            v_h = v_ref[:, h, :]
            wo_h = wo_ref[h * DH:(h + 1) * DH, :]
            for qb in range(SQ // QBLK):
                q0 = qb * QBLK
                q_blk = q_ref[q0:q0 + QBLK, h * DH:(h + 1) * DH]
                s = lax.dot_general(
                    q_blk, k_h, (((1,), (1,)), ((), ())),
                    preferred_element_type=jnp.float32) * SCALE
                rows = q0 + lax.broadcasted_iota(jnp.int32, (QBLK, SQ), 0)
                cols = lax.broadcasted_iota(jnp.int32, (QBLK, SQ), 1)
                qblock = rows // 64
                kblock = cols // 64
                mask = ((qblock == kblock) | (kblock == 0)
                        | (((qblock + kblock) % 3) == 0))
                s = jnp.where(mask, s, -1e9)
                m = jnp.max(s, axis=1, keepdims=True)
                w = jnp.exp(s - m)
                w = w / jnp.sum(w, axis=1, keepdims=True)
                ctx = jnp.dot(w, v_h, preferred_element_type=jnp.float32)
                contrib = jnp.dot(ctx, wo_h,
                                  preferred_element_type=jnp.float32)
                if h == 0:
                    out_ref[q0:q0 + QBLK, :] = contrib
                else:
                    out_ref[q0:q0 + QBLK, :] += contrib

        for s in range(N_DEV - 1):
            if s == 0:
                send_chunk = lax.rem(my + N_DEV, N_DEV)
                src = out_ref.at[pl.ds(send_chunk * CHUNK, CHUNK)]
            else:
                src = comm_ref.at[s - 1]
            rdma = pltpu.make_async_remote_copy(
                src_ref=src,
                dst_ref=comm_ref.at[s],
                send_sem=rs_send.at[s],
                recv_sem=rs_recv.at[s],
                device_id=(right,),
                device_id_type=pl.DeviceIdType.MESH,
            )
            rdma.start()
            rdma.wait()
            recv_chunk = lax.rem(my - s - 1 + 2 * N_DEV, N_DEV)
            comm_ref[s, :, :] = (comm_ref[s, :, :]
                                 + out_ref[pl.ds(recv_chunk * CHUNK, CHUNK), :])

        fin = lax.rem(my + 1, N_DEV)
        out_ref[pl.ds(fin * CHUNK, CHUNK), :] = comm_ref[N_DEV - 2, :, :]

        for s in range(N_DEV - 1):
            c = lax.rem(my + 1 - s + 2 * N_DEV, N_DEV)
            off = c * CHUNK
            rdma = pltpu.make_async_remote_copy(
                src_ref=out_ref.at[pl.ds(off, CHUNK)],
                dst_ref=out_ref.at[pl.ds(off, CHUNK)],
                send_sem=ag_send.at[s],
                recv_sem=ag_recv.at[s],
                device_id=(right,),
                device_id_type=pl.DeviceIdType.MESH,
            )
            rdma.start()
            rdma.wait()

    out = pl.pallas_call(
        body,
        out_shape=jax.ShapeDtypeStruct((SQ, D_MODEL), jnp.float32),
        in_specs=[pl.BlockSpec(memory_space=pltpu.VMEM)] * 4,
        out_specs=pl.BlockSpec(memory_space=pltpu.VMEM),
        scratch_shapes=[
            pltpu.VMEM((N_DEV - 1, CHUNK, D_MODEL), jnp.float32),
            pltpu.SemaphoreType.DMA((N_DEV - 1,)),
            pltpu.SemaphoreType.DMA((N_DEV - 1,)),
            pltpu.SemaphoreType.DMA((N_DEV - 1,)),
            pltpu.SemaphoreType.DMA((N_DEV - 1,)),
        ],
        compiler_params=pltpu.CompilerParams(
            collective_id=0,
            vmem_limit_bytes=100 * 1024 * 1024,
        ),
    )(Q, K, V, Wo_loc)
    return out[None]
